# Initial kernel scaffold; baseline (speedup 1.0000x reference)
import jax
import jax.numpy as jnp
from jax.experimental import pallas as pl


def kernel(x, f0, t0):
    raise NotImplementedError("write your pallas kernel here")


# speedup vs baseline: 1.1303x; 1.1303x over previous
import jax
import jax.numpy as jnp
from jax.experimental import pallas as pl
from jax.experimental.pallas import tpu as pltpu

_B, _F, _T = 64, 128, 4096
_HS = 64


def _body(f0_ref, t0_ref, x_ref, o_ref):
    b = pl.program_id(0)
    fi = jax.lax.broadcasted_iota(jnp.int32, (_F, _T), 0)
    ti = jax.lax.broadcasted_iota(jnp.int32, (_F, _T), 1)
    x = x_ref[0]
    f0a, t0a = f0_ref[b, 0], t0_ref[b, 0]
    f0b, t0b = f0_ref[b, 1], t0_ref[b, 1]
    m0 = (fi >= f0a) & (fi < f0a + _HS) & (ti >= t0a) & (ti < t0a + _HS)
    m1 = (fi >= f0b) & (fi < f0b + _HS) & (ti >= t0b) & (ti < t0b + _HS)
    o_ref[0] = jnp.where(m0 | m1, jnp.zeros((), x.dtype), x)


def kernel(x, f0, t0):
    grid_spec = pltpu.PrefetchScalarGridSpec(
        num_scalar_prefetch=2,
        grid=(_B,),
        in_specs=[pl.BlockSpec((1, _F, _T), lambda b, *_: (b, 0, 0))],
        out_specs=pl.BlockSpec((1, _F, _T), lambda b, *_: (b, 0, 0)),
    )
    return pl.pallas_call(
        _body,
        grid_spec=grid_spec,
        out_shape=jax.ShapeDtypeStruct(x.shape, x.dtype),
    )(f0.astype(jnp.int32), t0.astype(jnp.int32), x)


# speedup vs baseline: 1.2804x; 1.1328x over previous
import jax
import jax.numpy as jnp
from jax.experimental import pallas as pl
from jax.experimental.pallas import tpu as pltpu

_B, _F, _T = 64, 128, 4096
_HS = 64


def _body(f0_ref, t0_ref, x_ref, o_ref):
    b = pl.program_id(0)
    fi = jax.lax.broadcasted_iota(jnp.int32, (_F, 1), 0)
    ti = jax.lax.broadcasted_iota(jnp.int32, (1, _T), 1)
    x = x_ref[0]
    f0a, t0a = f0_ref[b, 0], t0_ref[b, 0]
    f0b, t0b = f0_ref[b, 1], t0_ref[b, 1]
    fm0 = (fi >= f0a) & (fi < f0a + _HS)
    tm0 = (ti >= t0a) & (ti < t0a + _HS)
    fm1 = (fi >= f0b) & (fi < f0b + _HS)
    tm1 = (ti >= t0b) & (ti < t0b + _HS)
    m = (fm0 & tm0) | (fm1 & tm1)
    o_ref[0] = jnp.where(m, jnp.zeros((), x.dtype), x)


def kernel(x, f0, t0):
    grid_spec = pltpu.PrefetchScalarGridSpec(
        num_scalar_prefetch=2,
        grid=(_B,),
        in_specs=[pl.BlockSpec((1, _F, _T), lambda b, *_: (b, 0, 0))],
        out_specs=pl.BlockSpec((1, _F, _T), lambda b, *_: (b, 0, 0)),
    )
    return pl.pallas_call(
        _body,
        grid_spec=grid_spec,
        out_shape=jax.ShapeDtypeStruct(x.shape, x.dtype),
    )(f0.astype(jnp.int32), t0.astype(jnp.int32), x)
